# baseline (speedup 1.0000x reference)
import functools

import jax
import jax.numpy as jnp
from jax import lax
from jax.experimental import pallas as pl
from jax.experimental.pallas import tpu as pltpu
from jax.experimental.pallas import tpu_sc as plsc

_NUM_CORES = 2
_NUM_SUBCORES = 16
_NUM_WORKERS = _NUM_CORES * _NUM_SUBCORES

_CHUNK = 400
_NBUF = 2


def _embed_kernel(n_pairs, table_hbm, idx_hbm, out_hbm, idx_v, bufs, gsems, ssems):
    wid = lax.axis_index("s") * _NUM_CORES + lax.axis_index("c")
    per_w = n_pairs // _NUM_WORKERS
    base = wid * per_w
    n_chunks = per_w // _CHUNK
    n_chunks = 1

    pltpu.sync_copy(idx_hbm.at[pl.ds(base, per_w)], idx_v)

    def fire_gather(i, b):
        idx_slice = idx_v.at[pl.ds(i * _CHUNK, _CHUNK)]
        return pltpu.async_copy(table_hbm.at[idx_slice], bufs[b], gsems[b])

    def fire_scatter(i, b):
        return pltpu.async_copy(
            bufs[b], out_hbm.at[pl.ds(base + i * _CHUNK, _CHUNK)], ssems[b])

    gcp = [None] * _NBUF
    scp = [None] * _NBUF
    for j in range(min(_NBUF, n_chunks)):
        gcp[j] = fire_gather(j, j)
    for i in range(n_chunks):
        b = i % _NBUF
        gcp[b].wait()
        scp[b] = fire_scatter(i, b)
        nxt = i + _NBUF
        if nxt < n_chunks:
            scp[b].wait()
            gcp[b] = fire_gather(nxt, b)
    for j in range(max(0, n_chunks - _NBUF), n_chunks):
        b = j % _NBUF
        if scp[b] is not None:
            scp[b].wait()
            scp[b] = None


def kernel(channel_names, table):
    b, s = channel_names.shape
    v, d = table.shape
    n_total = b * s
    n_pairs = n_total // 2

    pair_table = jnp.zeros((v * v, 2 * d), jnp.float32)
    pair_idx = jnp.zeros((n_pairs,), jnp.int32)

    mesh = plsc.VectorSubcoreMesh(
        core_axis_name="c", subcore_axis_name="s",
        num_cores=_NUM_CORES, num_subcores=_NUM_SUBCORES)

    gather = pl.kernel(
        functools.partial(_embed_kernel, n_pairs),
        out_type=jax.ShapeDtypeStruct((n_pairs, 2 * d), jnp.float32),
        mesh=mesh,
        scratch_types=[
            pltpu.VMEM((n_pairs // _NUM_WORKERS,), jnp.int32),
            [pltpu.VMEM((_CHUNK, 2 * d), jnp.float32) for _ in range(_NBUF)],
            [pltpu.SemaphoreType.DMA for _ in range(_NBUF)],
            [pltpu.SemaphoreType.DMA for _ in range(_NBUF)],
        ],
    )
    rows = gather(pair_table, pair_idx)
    embeddings = rows.reshape(b, s, d)
    padding_mask = jnp.zeros((b, s), dtype=bool)
    return (embeddings, padding_mask)


# speedup vs baseline: 1.3816x; 1.3816x over previous
import functools

import jax
import jax.numpy as jnp
from jax import lax
from jax.experimental import pallas as pl
from jax.experimental.pallas import tpu as pltpu
from jax.experimental.pallas import tpu_sc as plsc

_NUM_CORES = 2
_NUM_SUBCORES = 16
_NUM_WORKERS = _NUM_CORES * _NUM_SUBCORES

_RING = 8
_LANES = 16


def _embed_kernel(batch, seq, d, table_hbm, idx_hbm, out_hbm,
                  table_v, idx_v, s_buf, sems):
    wid = lax.axis_index("s") * _NUM_CORES + lax.axis_index("c")
    samples_per_w = batch // _NUM_WORKERS
    sample_base = wid * samples_per_w
    idx_base = sample_base * seq
    n_groups = samples_per_w // _RING

    pltpu.sync_copy(table_hbm, table_v)
    pltpu.sync_copy(idx_hbm.at[pl.ds(idx_base, samples_per_w * seq)],
                    idx_v.at[pl.ds(0, samples_per_w * seq)])

    lane = lax.iota(jnp.int32, _LANES)
    n_jblk = (seq + _LANES - 1) // _LANES

    def group_body(g, _):
        for k in range(_RING):
            smp = g * _RING + k
            dst = s_buf.at[k]

            @pl.when(g > 0)
            def _wait():
                pltpu.make_async_copy(s_buf.at[k], out_hbm.at[0],
                                      sems[k]).wait()

            def copy_pos(idx16, l, j):
                row = idx16[l]
                for c0 in range(d // _LANES):
                    dst[j, pl.ds(c0 * _LANES, _LANES)] = (
                        table_v[row, pl.ds(c0 * _LANES, _LANES)])

            def jblk_body(jb, _):
                idx16 = idx_v[pl.ds(smp * seq + jb * _LANES, _LANES)]
                for l in range(_LANES):
                    copy_pos(idx16, l, jb * _LANES + l)
                return _

            lax.fori_loop(0, seq // _LANES, jblk_body, 0)

            tail = seq - (seq % _LANES)
            if seq % _LANES:
                idx16 = idx_v[pl.ds(smp * seq + tail, _LANES)]
                for l in range(seq % _LANES):
                    copy_pos(idx16, l, tail + l)

            pltpu.async_copy(s_buf.at[k], out_hbm.at[sample_base + smp],
                             sems[k])
        return _

    lax.fori_loop(0, n_groups, group_body, 0)
    for k in range(_RING):
        pltpu.make_async_copy(s_buf.at[k], out_hbm.at[0], sems[k]).wait()


def kernel(channel_names, table):
    b, s = channel_names.shape
    v, d = table.shape
    n_total = b * s

    idx_flat = channel_names.reshape(n_total)

    mesh = plsc.VectorSubcoreMesh(
        core_axis_name="c", subcore_axis_name="s",
        num_cores=_NUM_CORES, num_subcores=_NUM_SUBCORES)

    per_w = n_total // _NUM_WORKERS

    gather = pl.kernel(
        functools.partial(_embed_kernel, b, s, d),
        out_type=jax.ShapeDtypeStruct((b, s, d), jnp.float32),
        mesh=mesh,
        scratch_types=[
            pltpu.VMEM((v, d), jnp.float32),
            pltpu.VMEM((per_w + _LANES, ), jnp.int32),
            pltpu.VMEM((_RING, s, d), jnp.float32),
            [pltpu.SemaphoreType.DMA for _ in range(_RING)],
        ],
    )
    embeddings = gather(table, idx_flat)
    padding_mask = jnp.zeros((b, s), dtype=bool)
    return (embeddings, padding_mask)


# speedup vs baseline: 2.0626x; 1.4928x over previous
import functools

import jax
import jax.numpy as jnp
from jax import lax
from jax.experimental import pallas as pl
from jax.experimental.pallas import tpu as pltpu
from jax.experimental.pallas import tpu_sc as plsc

_NUM_CORES = 2
_NUM_SUBCORES = 16
_NUM_WORKERS = _NUM_CORES * _NUM_SUBCORES

_RING = 8
_LANES = 16


def _embed_kernel(batch, seq, d, table_hbm, idx_hbm, out_hbm,
                  table_v, idx_v, s_buf, sems):
    wid = lax.axis_index("s") * _NUM_CORES + lax.axis_index("c")
    samples_per_w = batch // _NUM_WORKERS
    sample_base = wid * samples_per_w
    idx_base = sample_base * seq
    n_groups = samples_per_w // _RING

    pltpu.sync_copy(table_hbm, table_v)
    pltpu.sync_copy(idx_hbm.at[pl.ds(idx_base, samples_per_w * seq)],
                    idx_v.at[pl.ds(0, samples_per_w * seq)])

    lane = lax.iota(jnp.int32, _LANES)
    n_jblk = (seq + _LANES - 1) // _LANES

    def group_body(g, _):
        for k in range(_RING):
            smp = g * _RING + k
            dst = s_buf.at[k]

            @pl.when(g > 0)
            def _wait():
                pltpu.make_async_copy(s_buf.at[k], out_hbm.at[0],
                                      sems[k]).wait()

            def copy_block(j0, nl):
                idx16 = idx_v[pl.ds(smp * seq + j0, _LANES)]
                rows = [idx16[l] for l in range(nl)]
                for c0 in range(d // _LANES):
                    sl = pl.ds(c0 * _LANES, _LANES)
                    vals = [table_v[rows[l], sl] for l in range(nl)]
                    for l in range(nl):
                        dst[j0 + l, sl] = vals[l]

            def jblk_body(jb, _):
                copy_block(jb * _LANES, _LANES)
                return _

            lax.fori_loop(0, seq // _LANES, jblk_body, 0)

            if seq % _LANES:
                copy_block(seq - (seq % _LANES), seq % _LANES)

            pltpu.async_copy(s_buf.at[k], out_hbm.at[sample_base + smp],
                             sems[k])
        return _

    lax.fori_loop(0, n_groups, group_body, 0)
    for k in range(_RING):
        pltpu.make_async_copy(s_buf.at[k], out_hbm.at[0], sems[k]).wait()


def kernel(channel_names, table):
    b, s = channel_names.shape
    v, d = table.shape
    n_total = b * s

    idx_flat = channel_names.reshape(n_total)

    mesh = plsc.VectorSubcoreMesh(
        core_axis_name="c", subcore_axis_name="s",
        num_cores=_NUM_CORES, num_subcores=_NUM_SUBCORES)

    per_w = n_total // _NUM_WORKERS

    gather = pl.kernel(
        functools.partial(_embed_kernel, b, s, d),
        out_type=jax.ShapeDtypeStruct((b, s, d), jnp.float32),
        mesh=mesh,
        scratch_types=[
            pltpu.VMEM((v, d), jnp.float32),
            pltpu.VMEM((per_w + _LANES, ), jnp.int32),
            pltpu.VMEM((_RING, s, d), jnp.float32),
            [pltpu.SemaphoreType.DMA for _ in range(_RING)],
        ],
    )
    embeddings = gather(table, idx_flat)
    padding_mask = jnp.zeros((b, s), dtype=bool)
    return (embeddings, padding_mask)


# speedup vs baseline: 2.0727x; 1.0049x over previous
import functools

import jax
import jax.numpy as jnp
from jax import lax
from jax.experimental import pallas as pl
from jax.experimental.pallas import tpu as pltpu
from jax.experimental.pallas import tpu_sc as plsc

_NUM_CORES = 2
_NUM_SUBCORES = 16
_NUM_WORKERS = _NUM_CORES * _NUM_SUBCORES

_RING = 8
_LANES = 16


def _embed_kernel(batch, seq, d, table_hbm, idx_hbm, out_hbm,
                  table_v, idx_v, s_buf, sems):
    wid = lax.axis_index("s") * _NUM_CORES + lax.axis_index("c")
    samples_per_w = batch // _NUM_WORKERS
    sample_base = wid * samples_per_w
    idx_base = sample_base * seq
    n_groups = samples_per_w // _RING

    pltpu.sync_copy(table_hbm, table_v)
    pltpu.sync_copy(idx_hbm.at[pl.ds(idx_base, samples_per_w * seq)],
                    idx_v.at[pl.ds(0, samples_per_w * seq)])

    lane = lax.iota(jnp.int32, _LANES)
    n_jblk = (seq + _LANES - 1) // _LANES

    def group_body(g, _):
        for k in range(_RING):
            smp = g * _RING + k
            dst = s_buf.at[k]

            @pl.when(g > 0)
            def _wait():
                pltpu.make_async_copy(s_buf.at[k], out_hbm.at[0],
                                      sems[k]).wait()

            def copy_block(j0, nl):
                idx16 = idx_v[pl.ds(smp * seq + j0, _LANES)]
                rows = [idx16[l] for l in range(nl)]
                n_c = d // _LANES
                sls = [pl.ds(c0 * _LANES, _LANES) for c0 in range(n_c)]
                vals = [table_v[rows[l], sls[0]] for l in range(nl)]
                for c0 in range(n_c):
                    if c0 + 1 < n_c:
                        nxt = [table_v[rows[l], sls[c0 + 1]]
                               for l in range(nl)]
                    for l in range(nl):
                        dst[j0 + l, sls[c0]] = vals[l]
                    if c0 + 1 < n_c:
                        vals = nxt

            def jblk_body(jb, _):
                copy_block(jb * _LANES, _LANES)
                return _

            lax.fori_loop(0, seq // _LANES, jblk_body, 0)

            if seq % _LANES:
                copy_block(seq - (seq % _LANES), seq % _LANES)

            pltpu.async_copy(s_buf.at[k], out_hbm.at[sample_base + smp],
                             sems[k])
        return _

    lax.fori_loop(0, n_groups, group_body, 0)
    for k in range(_RING):
        pltpu.make_async_copy(s_buf.at[k], out_hbm.at[0], sems[k]).wait()


def kernel(channel_names, table):
    b, s = channel_names.shape
    v, d = table.shape
    n_total = b * s

    idx_flat = channel_names.reshape(n_total)

    mesh = plsc.VectorSubcoreMesh(
        core_axis_name="c", subcore_axis_name="s",
        num_cores=_NUM_CORES, num_subcores=_NUM_SUBCORES)

    per_w = n_total // _NUM_WORKERS

    gather = pl.kernel(
        functools.partial(_embed_kernel, b, s, d),
        out_type=jax.ShapeDtypeStruct((b, s, d), jnp.float32),
        mesh=mesh,
        scratch_types=[
            pltpu.VMEM((v, d), jnp.float32),
            pltpu.VMEM((per_w + _LANES, ), jnp.int32),
            pltpu.VMEM((_RING, s, d), jnp.float32),
            [pltpu.SemaphoreType.DMA for _ in range(_RING)],
        ],
    )
    embeddings = gather(table, idx_flat)
    padding_mask = jnp.zeros((b, s), dtype=bool)
    return (embeddings, padding_mask)
